# Initial kernel scaffold; baseline (speedup 1.0000x reference)
import jax
import jax.numpy as jnp
from jax.experimental import pallas as pl


def kernel(xyz, x, conv_w, conv_b, gamma, beta):
    raise NotImplementedError("write your pallas kernel here")


# speedup vs baseline: 12.2962x; 12.2962x over previous
import functools

import jax
import jax.numpy as jnp
from jax import lax
from jax.experimental import pallas as pl
from jax.experimental.pallas import tpu as pltpu
from jax.experimental.pallas import tpu_sc as plsc

KNN = 16
EPSV = 1e-5


def _knn_body(xyzp_ref, xyzT_ref, idx_ref):
    b = pl.program_id(0)
    a = xyzp_ref[0]
    t = xyzT_ref[0]
    sq_n = jnp.sum(a * a, axis=1, keepdims=True)
    sq_m = jnp.sum(t * t, axis=0, keepdims=True)
    ab = jnp.dot(a, t, preferred_element_type=jnp.float32)
    d = sq_n + sq_m - 2.0 * ab
    R, N = d.shape
    iota = lax.broadcasted_iota(jnp.int32, (R, N), 1)
    cols = []
    for _ in range(KNN):
        cur = jnp.min(d, axis=1, keepdims=True)
        m = d <= cur
        idxm = jnp.where(m, iota, N)
        amin = jnp.min(idxm, axis=1, keepdims=True)
        cols.append(amin)
        d = jnp.where(iota == amin, jnp.inf, d)
    idx_ref[0] = jnp.concatenate(cols, axis=1) + b * N


def _knn_indices(xyzp, xyzT, R=256):
    B, N, _ = xyzp.shape
    return pl.pallas_call(
        _knn_body,
        grid=(B, N // R),
        in_specs=[
            pl.BlockSpec((1, R, 8), lambda b, r: (b, r, 0)),
            pl.BlockSpec((1, 8, N), lambda b, r: (b, 0, 0)),
        ],
        out_specs=pl.BlockSpec((1, R, KNN), lambda b, r: (b, r, 0)),
        out_shape=jax.ShapeDtypeStruct((B, N, KNN), jnp.int32),
    )(xyzp, xyzT)


def _make_gather_sum(BN, C):
    info = plsc.get_sparse_core_info()
    NW = info.num_cores * info.num_subcores
    P = BN // NW
    G = 32
    IT = P // G
    RPI = G * KNN
    IDX_ROWS = RPI // 128
    mesh = plsc.VectorSubcoreMesh(core_axis_name="c", subcore_axis_name="s")

    @functools.partial(
        pl.kernel,
        mesh=mesh,
        out_type=jax.ShapeDtypeStruct((BN, C), jnp.float32),
        scratch_types=[
            pltpu.VMEM((IDX_ROWS, 128), jnp.int32),
            pltpu.VMEM((RPI, C), jnp.float32),
            pltpu.VMEM((G, C), jnp.float32),
            pltpu.SemaphoreType.DMA,
        ],
    )
    def k(table_hbm, idx_hbm, out_hbm, idxv, rows_v, out_v, sem):
        cid = lax.axis_index("c")
        sid = lax.axis_index("s")
        wid = sid * info.num_cores + cid

        def body(i, carry):
            irow0 = wid * (P * KNN // 128) + i * IDX_ROWS
            pltpu.sync_copy(idx_hbm.at[pl.ds(irow0, IDX_ROWS)], idxv)
            copies = []
            for j in range(IDX_ROWS):
                copies.append(pltpu.async_copy(
                    table_hbm.at[idxv.at[j]],
                    rows_v.at[pl.ds(j * 128, 128)],
                    sem,
                ))
            for cp in copies:
                cp.wait()

            def psum(g, c2):
                base = g * KNN
                for c in range(C // 16):
                    sl = pl.ds(c * 16, 16)
                    acc = rows_v[base, sl]
                    for j in range(1, KNN):
                        acc = acc + rows_v[base + j, sl]
                    out_v[g, sl] = acc
                return c2

            lax.fori_loop(0, G, psum, 0)
            pltpu.sync_copy(out_v, out_hbm.at[pl.ds(wid * P + i * G, G)])
            return carry

        lax.fori_loop(0, IT, body, 0)

    return k


def _conv_bn_body(BN, summed_ref, xt_ref, wT_ref, cb_ref, gam_ref, bet_ref,
                  out_ref, acc_ref):
    ph = pl.program_id(0)
    t = pl.program_id(1)

    dx = summed_ref[...] - xt_ref[...]
    h = jnp.dot(dx, wT_ref[...], preferred_element_type=jnp.float32) + cb_ref[...]
    h = jnp.maximum(h, 0.0)

    @pl.when(ph == 0)
    def _():
        @pl.when(t == 0)
        def _():
            acc_ref[...] = jnp.zeros_like(acc_ref)
        acc_ref[0:1, :] += jnp.sum(h, axis=0, keepdims=True)
        acc_ref[1:2, :] += jnp.sum(h * h, axis=0, keepdims=True)

    @pl.when(ph == 1)
    def _():
        inv_n = 1.0 / BN
        mean = acc_ref[0:1, :] * inv_n
        var = acc_ref[1:2, :] * inv_n - mean * mean
        rstd = lax.rsqrt(var + EPSV)
        hn = (h - mean) * rstd
        out_ref[...] = xt_ref[...] + gam_ref[...] * hn + bet_ref[...]


def _conv_bn(summed, xt, conv_wT, cb2, gam2, bet2, R2=512):
    BN, C = summed.shape
    T = BN // R2
    return pl.pallas_call(
        functools.partial(_conv_bn_body, BN),
        grid=(2, T),
        in_specs=[
            pl.BlockSpec((R2, C), lambda p, t: (t, 0)),
            pl.BlockSpec((R2, C), lambda p, t: (t, 0)),
            pl.BlockSpec((C, C), lambda p, t: (0, 0)),
            pl.BlockSpec((1, C), lambda p, t: (0, 0)),
            pl.BlockSpec((1, C), lambda p, t: (0, 0)),
            pl.BlockSpec((1, C), lambda p, t: (0, 0)),
        ],
        out_specs=pl.BlockSpec((R2, C), lambda p, t: (t, 0)),
        out_shape=jax.ShapeDtypeStruct((BN, C), jnp.float32),
        scratch_shapes=[pltpu.VMEM((8, C), jnp.float32)],
    )(summed, xt, conv_wT, cb2, gam2, bet2)


def kernel(xyz, x, conv_w, conv_b, gamma, beta):
    B, C, N = x.shape
    BN = B * N

    xyzp = jnp.pad(xyz, ((0, 0), (0, 0), (0, 5)))
    xyzT = jnp.transpose(xyzp, (0, 2, 1))
    x_t = jnp.transpose(x, (0, 2, 1)).reshape(BN, C)

    idx = _knn_indices(xyzp, xyzT)
    idx2d = idx.reshape(BN * KNN // 128, 128)

    summed = _make_gather_sum(BN, C)(x_t, idx2d)

    out_t = _conv_bn(
        summed, x_t, conv_w.T,
        conv_b.reshape(1, C), gamma.reshape(1, C), beta.reshape(1, C),
    )
    return jnp.transpose(out_t.reshape(B, N, C), (0, 2, 1))


# speedup vs baseline: 16.3234x; 1.3275x over previous
import functools

import jax
import jax.numpy as jnp
from jax import lax
from jax.experimental import pallas as pl
from jax.experimental.pallas import tpu as pltpu
from jax.experimental.pallas import tpu_sc as plsc

KNN = 16
EPSV = 1e-5


def _knn_body(xyzp_ref, xyzT_ref, idx_ref):
    b = pl.program_id(0)
    a = xyzp_ref[0]
    t = xyzT_ref[0]
    sq_n = jnp.sum(a * a, axis=1, keepdims=True)
    sq_m = jnp.sum(t * t, axis=0, keepdims=True)
    ab = jnp.dot(a, t, preferred_element_type=jnp.float32)
    d = sq_n + sq_m - 2.0 * ab
    R, N = d.shape
    iota_f = lax.broadcasted_iota(jnp.int32, (R, N), 1).astype(jnp.float32)
    big = float(2 * N)
    cols = []
    for _ in range(KNN):
        cur = jnp.min(d, axis=1, keepdims=True)
        m = d <= cur
        amin = jnp.min(jnp.where(m, iota_f, big), axis=1, keepdims=True)
        cols.append(amin)
        d = jnp.where(iota_f == amin, jnp.inf, d)
    idx = jnp.concatenate(cols, axis=1).astype(jnp.int32)
    idx_ref[0] = idx + b * N


def _knn_indices(xyzp, xyzT, R=256):
    B, N, _ = xyzp.shape
    return pl.pallas_call(
        _knn_body,
        grid=(B, N // R),
        in_specs=[
            pl.BlockSpec((1, R, 8), lambda b, r: (b, r, 0)),
            pl.BlockSpec((1, 8, N), lambda b, r: (b, 0, 0)),
        ],
        out_specs=pl.BlockSpec((1, R, KNN), lambda b, r: (b, r, 0)),
        out_shape=jax.ShapeDtypeStruct((B, N, KNN), jnp.int32),
    )(xyzp, xyzT)


def _make_gather_sum(BN, C):
    info = plsc.get_sparse_core_info()
    NW = info.num_cores * info.num_subcores
    P = BN // NW
    G = 16
    IT = P // G
    HALF = IT // 2
    RPI = G * KNN
    IDX_ROWS = RPI // 128
    NGATH = RPI // 128
    mesh = plsc.VectorSubcoreMesh(core_axis_name="c", subcore_axis_name="s")

    @functools.partial(
        pl.kernel,
        mesh=mesh,
        out_type=jax.ShapeDtypeStruct((BN, C), jnp.float32),
        scratch_types=[
            pltpu.VMEM((2, IDX_ROWS, 128), jnp.int32),
            pltpu.VMEM((2, RPI, C), jnp.float32),
            pltpu.VMEM((2, G, C), jnp.float32),
            pltpu.SemaphoreType.DMA,
            pltpu.SemaphoreType.DMA,
            pltpu.SemaphoreType.DMA,
            pltpu.SemaphoreType.DMA,
        ],
    )
    def k(table_hbm, idx_hbm, out_hbm, idxv, rows_v, out_v, sg0, sg1, so0, so1):
        cid = lax.axis_index("c")
        sid = lax.axis_index("s")
        wid = sid * info.num_cores + cid
        gsems = (sg0, sg1)
        osems = (so0, so1)

        def stage(i, slot):
            irow0 = wid * (P * KNN // 128) + i * IDX_ROWS
            pltpu.sync_copy(idx_hbm.at[pl.ds(irow0, IDX_ROWS)], idxv.at[slot])
            for j in range(NGATH):
                pltpu.async_copy(
                    table_hbm.at[idxv.at[slot].at[j]],
                    rows_v.at[slot].at[pl.ds(j * 128, 128)],
                    gsems[slot],
                )

        def wait_gathers(slot):
            for j in range(NGATH):
                pltpu.make_async_copy(
                    table_hbm.at[idxv.at[slot].at[j]],
                    rows_v.at[slot].at[pl.ds(j * 128, 128)],
                    gsems[slot],
                ).wait()

        def compute_out(i, slot, first):
            rv = rows_v.at[slot]
            ov = out_v.at[slot]

            def psum(g, c2):
                base = g * KNN
                for c in range(C // 16):
                    sl = pl.ds(c * 16, 16)
                    acc = rv[base, sl]
                    for j in range(1, KNN):
                        acc = acc + rv[base + j, sl]
                    ov[g, sl] = acc
                return c2

            dst = out_hbm.at[pl.ds(wid * P + i * G, G)]
            if first:
                lax.fori_loop(0, G, psum, 0)
            else:
                pltpu.make_async_copy(ov, dst, osems[slot]).wait()
                lax.fori_loop(0, G, psum, 0)
            pltpu.async_copy(ov, dst, osems[slot])

        stage(jnp.int32(0), 0)

        def body(jj, carry):
            i0 = 2 * jj
            stage(i0 + 1, 1)
            wait_gathers(0)

            @pl.when(jj == 0)
            def _():
                compute_out(i0, 0, True)

            @pl.when(jj > 0)
            def _():
                compute_out(i0, 0, False)

            @pl.when(jj + 1 < HALF)
            def _():
                stage(i0 + 2, 0)

            wait_gathers(1)

            @pl.when(jj == 0)
            def _():
                compute_out(i0 + 1, 1, True)

            @pl.when(jj > 0)
            def _():
                compute_out(i0 + 1, 1, False)

            return carry

        lax.fori_loop(0, HALF, body, 0)
        for slot in range(2):
            i_last = IT - 2 + slot
            pltpu.make_async_copy(
                out_v.at[slot],
                out_hbm.at[pl.ds(wid * P + i_last * G, G)],
                osems[slot],
            ).wait()

    return k


def _conv_bn_body(BN, TN, summed_ref, x_ref, w_ref, cb_ref, gam_ref, bet_ref,
                  out_ref, acc_ref):
    ph = pl.program_id(0)
    b = pl.program_id(1)
    t = pl.program_id(2)

    xb = x_ref[0]
    s_t = summed_ref[...]
    w = w_ref[...]
    h = lax.dot_general(w, s_t, (((1,), (1,)), ((), ())),
                        preferred_element_type=jnp.float32)
    h = h - jnp.dot(w, xb, preferred_element_type=jnp.float32)
    h = h + cb_ref[...]
    h = jnp.maximum(h, 0.0)

    @pl.when(ph == 0)
    def _():
        @pl.when((b == 0) & (t == 0))
        def _():
            acc_ref[...] = jnp.zeros_like(acc_ref)
        acc_ref[:, 0:1] += jnp.sum(h, axis=1, keepdims=True)
        acc_ref[:, 1:2] += jnp.sum(h * h, axis=1, keepdims=True)

    @pl.when(ph == 1)
    def _():
        inv_n = 1.0 / BN
        mean = acc_ref[:, 0:1] * inv_n
        var = acc_ref[:, 1:2] * inv_n - mean * mean
        rstd = lax.rsqrt(var + EPSV)
        hn = (h - mean) * rstd
        out_ref[0] = xb + gam_ref[...] * hn + bet_ref[...]


def _conv_bn(summed, x, conv_w, cb2, gam2, bet2, R2=512):
    B, C, N = x.shape
    BN = B * N
    TN = N // R2
    return pl.pallas_call(
        functools.partial(_conv_bn_body, BN, TN),
        grid=(2, B, TN),
        in_specs=[
            pl.BlockSpec((R2, C), lambda p, b, t: (b * TN + t, 0)),
            pl.BlockSpec((1, C, R2), lambda p, b, t: (b, 0, t)),
            pl.BlockSpec((C, C), lambda p, b, t: (0, 0)),
            pl.BlockSpec((C, 1), lambda p, b, t: (0, 0)),
            pl.BlockSpec((C, 1), lambda p, b, t: (0, 0)),
            pl.BlockSpec((C, 1), lambda p, b, t: (0, 0)),
        ],
        out_specs=pl.BlockSpec((1, C, R2), lambda p, b, t: (b, 0, t)),
        out_shape=jax.ShapeDtypeStruct((B, C, N), jnp.float32),
        scratch_shapes=[pltpu.VMEM((C, 128), jnp.float32)],
    )(summed, x, conv_w, cb2, gam2, bet2)


def kernel(xyz, x, conv_w, conv_b, gamma, beta):
    B, C, N = x.shape
    BN = B * N

    xyzp = jnp.pad(xyz, ((0, 0), (0, 0), (0, 5)))
    xyzT = jnp.transpose(xyzp, (0, 2, 1))
    x_t = jnp.transpose(x, (0, 2, 1)).reshape(BN, C)

    idx = _knn_indices(xyzp, xyzT)
    idx2d = idx.reshape(BN * KNN // 128, 128)

    summed = _make_gather_sum(BN, C)(x_t, idx2d)

    return _conv_bn(
        summed, x, conv_w,
        conv_b.reshape(C, 1), gamma.reshape(C, 1), beta.reshape(C, 1),
    )


# speedup vs baseline: 18.9336x; 1.1599x over previous
import functools

import jax
import jax.numpy as jnp
from jax import lax
from jax.experimental import pallas as pl
from jax.experimental.pallas import tpu as pltpu
from jax.experimental.pallas import tpu_sc as plsc

KNN = 16
EPSV = 1e-5


def _knn_body(base, xyz_ref, xyzT_ref, idx_ref):
    b = pl.program_id(0)
    a = xyz_ref[0]
    t = xyzT_ref[0]
    sq_n = jnp.sum(a * a, axis=1, keepdims=True)
    sq_m = jnp.sum(t * t, axis=0, keepdims=True)
    ab = jnp.dot(a, t, preferred_element_type=jnp.float32)
    d = sq_n + sq_m - 2.0 * ab
    R, N = d.shape
    iota_f = lax.broadcasted_iota(jnp.int32, (R, N), 1).astype(jnp.float32)
    big = float(2 * N)
    cols = []
    for _ in range(KNN):
        cur = jnp.min(d, axis=1, keepdims=True)
        m = d <= cur
        amin = jnp.min(jnp.where(m, iota_f, big), axis=1, keepdims=True)
        cols.append(amin)
        d = jnp.where(iota_f == amin, jnp.inf, d)
    idx = jnp.concatenate(cols, axis=1).astype(jnp.int32) + (b * N + base)
    idx_ref[0] = idx


def _knn_indices(xyz, xyzT, base, R=256):
    B, N, _ = xyz.shape
    return pl.pallas_call(
        functools.partial(_knn_body, base),
        grid=(B, N // R),
        in_specs=[
            pl.BlockSpec((1, R, 8), lambda b, r: (b, r, 0)),
            pl.BlockSpec((1, 8, N), lambda b, r: (b, 0, 0)),
        ],
        out_specs=pl.BlockSpec((1, R, KNN), lambda b, r: (b, r, 0)),
        out_shape=jax.ShapeDtypeStruct((B, N, KNN), jnp.int32),
    )(xyz, xyzT)


def _make_gather_sum(CN, C):
    info = plsc.get_sparse_core_info()
    NW = info.num_cores * info.num_subcores
    P = CN // NW
    G = 16
    IT = P // G
    HALF = IT // 2
    RPI = G * KNN
    NGATH = RPI // 128
    WROWS = P * KNN // 128
    mesh = plsc.VectorSubcoreMesh(core_axis_name="c", subcore_axis_name="s")

    @functools.partial(
        pl.kernel,
        mesh=mesh,
        out_type=jax.ShapeDtypeStruct((CN, C), jnp.float32),
        scratch_types=[
            pltpu.VMEM((WROWS, 128), jnp.int32),
            pltpu.VMEM((2, RPI, C), jnp.float32),
            pltpu.VMEM((2, G, C), jnp.float32),
            pltpu.SemaphoreType.DMA,
            pltpu.SemaphoreType.DMA,
            pltpu.SemaphoreType.DMA,
            pltpu.SemaphoreType.DMA,
        ],
    )
    def k(table_hbm, idx_hbm, out_hbm, idxv, rows_v, out_v, sg0, sg1, so0, so1):
        cid = lax.axis_index("c")
        sid = lax.axis_index("s")
        wid = sid * info.num_cores + cid
        gsems = (sg0, sg1)
        osems = (so0, so1)

        pltpu.sync_copy(idx_hbm.at[pl.ds(wid * WROWS, WROWS)], idxv)

        def stage(i, slot):
            for j in range(NGATH):
                pltpu.async_copy(
                    table_hbm.at[idxv.at[NGATH * i + j]],
                    rows_v.at[slot].at[pl.ds(j * 128, 128)],
                    gsems[slot],
                )

        def wait_gathers(i, slot):
            for j in range(NGATH):
                pltpu.make_async_copy(
                    table_hbm.at[idxv.at[NGATH * i + j]],
                    rows_v.at[slot].at[pl.ds(j * 128, 128)],
                    gsems[slot],
                ).wait()

        def compute_out(i, slot, first):
            rv = rows_v.at[slot]
            ov = out_v.at[slot]

            def psum(g, c2):
                base = g * KNN
                for c in range(C // 16):
                    sl = pl.ds(c * 16, 16)
                    acc = rv[base, sl]
                    for j in range(1, KNN):
                        acc = acc + rv[base + j, sl]
                    ov[g, sl] = acc
                return c2

            dst = out_hbm.at[pl.ds(wid * P + i * G, G)]
            if not first:
                pltpu.make_async_copy(ov, dst, osems[slot]).wait()
            lax.fori_loop(0, G, psum, 0)
            pltpu.async_copy(ov, dst, osems[slot])

        stage(jnp.int32(0), 0)

        def body(jj, carry):
            i0 = 2 * jj
            stage(i0 + 1, 1)
            wait_gathers(i0, 0)

            @pl.when(jj == 0)
            def _():
                compute_out(i0, 0, True)

            @pl.when(jj > 0)
            def _():
                compute_out(i0, 0, False)

            @pl.when(jj + 1 < HALF)
            def _():
                stage(i0 + 2, 0)

            wait_gathers(i0 + 1, 1)

            @pl.when(jj == 0)
            def _():
                compute_out(i0 + 1, 1, True)

            @pl.when(jj > 0)
            def _():
                compute_out(i0 + 1, 1, False)

            return carry

        lax.fori_loop(0, HALF, body, 0)
        for slot in range(2):
            i_last = IT - 2 + slot
            pltpu.make_async_copy(
                out_v.at[slot],
                out_hbm.at[pl.ds(wid * P + i_last * G, G)],
                osems[slot],
            ).wait()

    return k


def _conv_bn_body(BN, summed_ref, x_ref, w_ref, cb_ref, gam_ref, bet_ref,
                  out_ref, acc_ref):
    ph = pl.program_id(0)
    b = pl.program_id(1)
    t = pl.program_id(2)

    xb = x_ref[0]
    s_t = summed_ref[...]
    w = w_ref[...]
    h = lax.dot_general(w, s_t, (((1,), (1,)), ((), ())),
                        preferred_element_type=jnp.float32)
    h = h - jnp.dot(w, xb, preferred_element_type=jnp.float32)
    h = h + cb_ref[...]
    h = jnp.maximum(h, 0.0)

    @pl.when(ph == 0)
    def _():
        @pl.when((b == 0) & (t == 0))
        def _():
            acc_ref[...] = jnp.zeros_like(acc_ref)
        acc_ref[:, 0:1] += jnp.sum(h, axis=1, keepdims=True)
        acc_ref[:, 1:2] += jnp.sum(h * h, axis=1, keepdims=True)

    @pl.when(ph == 1)
    def _():
        inv_n = 1.0 / BN
        mean = acc_ref[:, 0:1] * inv_n
        var = acc_ref[:, 1:2] * inv_n - mean * mean
        rstd = lax.rsqrt(var + EPSV)
        hn = (h - mean) * rstd
        out_ref[0] = xb + gam_ref[...] * hn + bet_ref[...]


def _conv_bn(summed, x, conv_w, cb2, gam2, bet2, R2=1024):
    B, C, N = x.shape
    BN = B * N
    TN = N // R2
    return pl.pallas_call(
        functools.partial(_conv_bn_body, BN),
        grid=(2, B, TN),
        in_specs=[
            pl.BlockSpec((R2, C), lambda p, b, t: (b * TN + t, 0)),
            pl.BlockSpec((1, C, R2), lambda p, b, t: (b, 0, t)),
            pl.BlockSpec((C, C), lambda p, b, t: (0, 0)),
            pl.BlockSpec((C, 1), lambda p, b, t: (0, 0)),
            pl.BlockSpec((C, 1), lambda p, b, t: (0, 0)),
            pl.BlockSpec((C, 1), lambda p, b, t: (0, 0)),
        ],
        out_specs=pl.BlockSpec((1, C, R2), lambda p, b, t: (b, 0, t)),
        out_shape=jax.ShapeDtypeStruct((B, C, N), jnp.float32),
        scratch_shapes=[pltpu.VMEM((C, 128), jnp.float32)],
    )(summed, x, conv_w, cb2, gam2, bet2)


def kernel(xyz, x, conv_w, conv_b, gamma, beta):
    B, C, N = x.shape
    BN = B * N

    xyzp = jnp.pad(xyz, ((0, 0), (0, 0), (0, 5)))
    xyzT = jnp.transpose(xyzp, (0, 2, 1))
    x_t = jnp.transpose(x, (0, 2, 1)).reshape(BN, C)

    NCH = 4
    BC = B // NCH
    CN = BC * N
    gather = _make_gather_sum(CN, C)
    summeds = []
    for c in range(NCH):
        xp_c = lax.slice_in_dim(xyzp, c * BC, (c + 1) * BC, axis=0)
        xT_c = lax.slice_in_dim(xyzT, c * BC, (c + 1) * BC, axis=0)
        idx_c = _knn_indices(xp_c, xT_c, c * CN)
        idx2d_c = idx_c.reshape(CN * KNN // 128, 128)
        summeds.append(gather(x_t, idx2d_c))
    summed = jnp.concatenate(summeds, axis=0)

    return _conv_bn(
        summed, x, conv_w,
        conv_b.reshape(C, 1), gamma.reshape(C, 1), beta.reshape(C, 1),
    )


# speedup vs baseline: 19.7543x; 1.0433x over previous
import functools

import jax
import jax.numpy as jnp
from jax import lax
from jax.experimental import pallas as pl
from jax.experimental.pallas import tpu as pltpu
from jax.experimental.pallas import tpu_sc as plsc

KNN = 16
EPSV = 1e-5


def _knn_body(base, xyz_ref, xyzT_ref, idx_ref):
    b = pl.program_id(0)
    a = xyz_ref[0]
    t = xyzT_ref[0]
    sq_n = jnp.sum(a * a, axis=1, keepdims=True)
    sq_m = jnp.sum(t * t, axis=0, keepdims=True)
    ab = jnp.dot(a, t, preferred_element_type=jnp.float32)
    d = sq_n + sq_m - 2.0 * ab
    R, N = d.shape
    r = pl.program_id(1)
    goff = b * N + base
    iota_f = (lax.broadcasted_iota(jnp.int32, (R, N), 1) + goff).astype(jnp.float32)
    big = (goff + 2 * N).astype(jnp.float32)
    self_f = (lax.broadcasted_iota(jnp.int32, (R, 1), 0)
              + (goff + r * R)).astype(jnp.float32)
    cols = [self_f]
    d = jnp.where(iota_f == self_f, jnp.inf, d)
    for k in range(1, KNN):
        cur = jnp.min(d, axis=1, keepdims=True)
        m = d <= cur
        amin = jnp.min(jnp.where(m, iota_f, big), axis=1, keepdims=True)
        cols.append(amin)
        if k < KNN - 1:
            d = jnp.where(iota_f == amin, jnp.inf, d)
    idx_ref[0] = jnp.concatenate(cols, axis=1).astype(jnp.int32)


def _knn_indices(xyz, xyzT, base, R=256):
    B, N, _ = xyz.shape
    return pl.pallas_call(
        functools.partial(_knn_body, base),
        grid=(B, N // R),
        in_specs=[
            pl.BlockSpec((1, R, 8), lambda b, r: (b, r, 0)),
            pl.BlockSpec((1, 8, N), lambda b, r: (b, 0, 0)),
        ],
        out_specs=pl.BlockSpec((1, R, KNN), lambda b, r: (b, r, 0)),
        out_shape=jax.ShapeDtypeStruct((B, N, KNN), jnp.int32),
    )(xyz, xyzT)


def _make_gather_sum(CN, C):
    info = plsc.get_sparse_core_info()
    NW = info.num_cores * info.num_subcores
    P = CN // NW
    G = 16
    IT = P // G
    HALF = IT // 2
    RPI = G * KNN
    NGATH = RPI // 128
    WROWS = P * KNN // 128
    mesh = plsc.VectorSubcoreMesh(core_axis_name="c", subcore_axis_name="s")

    @functools.partial(
        pl.kernel,
        mesh=mesh,
        out_type=jax.ShapeDtypeStruct((CN, C), jnp.float32),
        scratch_types=[
            pltpu.VMEM((WROWS, 128), jnp.int32),
            pltpu.VMEM((2, RPI, C), jnp.float32),
            pltpu.VMEM((2, G, C), jnp.float32),
            pltpu.SemaphoreType.DMA,
            pltpu.SemaphoreType.DMA,
            pltpu.SemaphoreType.DMA,
            pltpu.SemaphoreType.DMA,
        ],
    )
    def k(table_hbm, idx_hbm, out_hbm, idxv, rows_v, out_v, sg0, sg1, so0, so1):
        cid = lax.axis_index("c")
        sid = lax.axis_index("s")
        wid = sid * info.num_cores + cid
        gsems = (sg0, sg1)
        osems = (so0, so1)

        pltpu.sync_copy(idx_hbm.at[pl.ds(wid * WROWS, WROWS)], idxv)

        def stage(i, slot):
            for j in range(NGATH):
                pltpu.async_copy(
                    table_hbm.at[idxv.at[NGATH * i + j]],
                    rows_v.at[slot].at[pl.ds(j * 128, 128)],
                    gsems[slot],
                )

        def wait_gathers(i, slot):
            for j in range(NGATH):
                pltpu.make_async_copy(
                    table_hbm.at[idxv.at[NGATH * i + j]],
                    rows_v.at[slot].at[pl.ds(j * 128, 128)],
                    gsems[slot],
                ).wait()

        def compute_out(i, slot, first):
            rv = rows_v.at[slot]
            ov = out_v.at[slot]

            def psum(g, c2):
                base = g * KNN
                for c in range(C // 16):
                    sl = pl.ds(c * 16, 16)
                    acc = rv[base, sl]
                    for j in range(1, KNN):
                        acc = acc + rv[base + j, sl]
                    ov[g, sl] = acc
                return c2

            dst = out_hbm.at[pl.ds(wid * P + i * G, G)]
            if not first:
                pltpu.make_async_copy(ov, dst, osems[slot]).wait()
            lax.fori_loop(0, G, psum, 0)
            pltpu.async_copy(ov, dst, osems[slot])

        stage(jnp.int32(0), 0)

        def body(jj, carry):
            i0 = 2 * jj
            stage(i0 + 1, 1)
            wait_gathers(i0, 0)

            @pl.when(jj == 0)
            def _():
                compute_out(i0, 0, True)

            @pl.when(jj > 0)
            def _():
                compute_out(i0, 0, False)

            @pl.when(jj + 1 < HALF)
            def _():
                stage(i0 + 2, 0)

            wait_gathers(i0 + 1, 1)

            @pl.when(jj == 0)
            def _():
                compute_out(i0 + 1, 1, True)

            @pl.when(jj > 0)
            def _():
                compute_out(i0 + 1, 1, False)

            return carry

        lax.fori_loop(0, HALF, body, 0)
        for slot in range(2):
            i_last = IT - 2 + slot
            pltpu.make_async_copy(
                out_v.at[slot],
                out_hbm.at[pl.ds(wid * P + i_last * G, G)],
                osems[slot],
            ).wait()

    return k


def _conv_bn_body(BN, summed_ref, x_ref, w_ref, cb_ref, gam_ref, bet_ref,
                  out_ref, acc_ref):
    ph = pl.program_id(0)
    b = pl.program_id(1)
    t = pl.program_id(2)

    xb = x_ref[0]
    s_t = summed_ref[...]
    w = w_ref[...]
    h = lax.dot_general(w, s_t, (((1,), (1,)), ((), ())),
                        preferred_element_type=jnp.float32)
    h = h - jnp.dot(w, xb, preferred_element_type=jnp.float32)
    h = h + cb_ref[...]
    h = jnp.maximum(h, 0.0)

    @pl.when(ph == 0)
    def _():
        @pl.when((b == 0) & (t == 0))
        def _():
            acc_ref[...] = jnp.zeros_like(acc_ref)
        acc_ref[:, 0:1] += jnp.sum(h, axis=1, keepdims=True)
        acc_ref[:, 1:2] += jnp.sum(h * h, axis=1, keepdims=True)

    @pl.when(ph == 1)
    def _():
        inv_n = 1.0 / BN
        mean = acc_ref[:, 0:1] * inv_n
        var = acc_ref[:, 1:2] * inv_n - mean * mean
        rstd = lax.rsqrt(var + EPSV)
        hn = (h - mean) * rstd
        out_ref[0] = xb + gam_ref[...] * hn + bet_ref[...]


def _conv_bn(summed, x, conv_w, cb2, gam2, bet2, R2=1024):
    B, C, N = x.shape
    BN = B * N
    TN = N // R2
    return pl.pallas_call(
        functools.partial(_conv_bn_body, BN),
        grid=(2, B, TN),
        in_specs=[
            pl.BlockSpec((R2, C), lambda p, b, t: (b * TN + t, 0)),
            pl.BlockSpec((1, C, R2), lambda p, b, t: (b, 0, t)),
            pl.BlockSpec((C, C), lambda p, b, t: (0, 0)),
            pl.BlockSpec((C, 1), lambda p, b, t: (0, 0)),
            pl.BlockSpec((C, 1), lambda p, b, t: (0, 0)),
            pl.BlockSpec((C, 1), lambda p, b, t: (0, 0)),
        ],
        out_specs=pl.BlockSpec((1, C, R2), lambda p, b, t: (b, 0, t)),
        out_shape=jax.ShapeDtypeStruct((B, C, N), jnp.float32),
        scratch_shapes=[pltpu.VMEM((C, 128), jnp.float32)],
    )(summed, x, conv_w, cb2, gam2, bet2)


def kernel(xyz, x, conv_w, conv_b, gamma, beta):
    B, C, N = x.shape
    BN = B * N

    xyzp = jnp.pad(xyz, ((0, 0), (0, 0), (0, 5)))
    xyzT = jnp.transpose(xyzp, (0, 2, 1))
    x_t = jnp.transpose(x, (0, 2, 1)).reshape(BN, C)

    NCH = 4
    BC = B // NCH
    CN = BC * N
    gather = _make_gather_sum(CN, C)
    summeds = []
    for c in range(NCH):
        xp_c = lax.slice_in_dim(xyzp, c * BC, (c + 1) * BC, axis=0)
        xT_c = lax.slice_in_dim(xyzT, c * BC, (c + 1) * BC, axis=0)
        idx_c = _knn_indices(xp_c, xT_c, c * CN)
        idx2d_c = idx_c.reshape(CN * KNN // 128, 128)
        summeds.append(gather(x_t, idx2d_c))
    summed = jnp.concatenate(summeds, axis=0)

    return _conv_bn(
        summed, x, conv_w,
        conv_b.reshape(C, 1), gamma.reshape(C, 1), beta.reshape(C, 1),
    )
